# baseline (speedup 1.0000x reference)
import jax
import jax.numpy as jnp
from jax import lax
from jax.experimental import pallas as pl
from jax.experimental.pallas import tpu as pltpu

EPS = 1e-6
_B = 512
_C = 256
_GP = 256
_SW = 128


def _row_mean(srow):
    return jnp.mean(srow, axis=1, keepdims=True)


def _seg_onehot(starts, ends, i):
    r = i * _B + lax.broadcasted_iota(jnp.int32, (_B, _GP), 0)
    return ((r >= starts[None, :]) & (r < ends[None, :])).astype(jnp.float32)


def _fused_kernel(starts_ref, ends_ref, splits_ref, s_ref, v_ref,
                  w_ref, b_ref, sout_ref, vout_ref,
                  s1_acc, s2_acc, vn_acc, gath, s0_scr, v_scr):
    i = pl.program_id(0)
    nb = pl.num_programs(0) // 2
    blk = jnp.where(i < nb, i, i - nb)
    onehot = _seg_onehot(starts_ref[0, :], ends_ref[0, :], blk)

    @pl.when(i == 0)
    def _init():
        s1_acc[...] = jnp.zeros_like(s1_acc)
        s2_acc[...] = jnp.zeros_like(s2_acc)
        vn_acc[...] = jnp.zeros_like(vn_acc)

    @pl.when(i < nb)
    def _stats():
        valid = (blk * _B + lax.broadcasted_iota(jnp.int32, (_B, 1), 0)) < \
            ends_ref[0, _GP - 1]
        srow = s_ref[...]
        s0 = jnp.where(valid, srow - _row_mean(srow), 0.0)
        s0 = s0.astype(jnp.bfloat16).astype(jnp.float32)
        vrow = v_ref[...]
        vnmat = jnp.sqrt(jnp.sum(vrow * vrow, axis=1) + EPS)
        vnmat = jnp.where(valid, vnmat, 0.0)
        s0_scr[pl.ds(blk * _B, _B), :] = s0.astype(jnp.bfloat16)
        v_scr[pl.ds(blk * _B, _B), :, :] = vrow.astype(jnp.bfloat16)
        dn = (((0,), (0,)), ((), ()))
        s1_acc[...] += lax.dot_general(
            onehot, s0, dn, precision=lax.Precision.HIGHEST,
            preferred_element_type=jnp.float32)
        s2_acc[...] += lax.dot_general(
            onehot, s0 * s0, dn, preferred_element_type=jnp.float32)
        vn_acc[...] += lax.dot_general(
            onehot, vnmat, dn, preferred_element_type=jnp.float32)

    @pl.when(i == nb - 1)
    def _finalize():
        counts = jnp.maximum(splits_ref[0, :], 1).astype(jnp.float32)
        means = s1_acc[...] / counts[:, None]
        var = (jnp.sum(s2_acc[...], axis=1) / counts
               - jnp.sum(means * means, axis=1)) / _C
        inv_std = 1.0 / jnp.sqrt(jnp.maximum(var, 0.0) + EPS)
        vnorm = jnp.sum(vn_acc[...], axis=1) / (counts * _C)
        inv_vn = jnp.where(vnorm > 0, 1.0 / vnorm, 0.0)
        gath[:, 0:_C] = means
        gath[:, _C:] = jnp.concatenate(
            [inv_std[:, None], inv_vn[:, None],
             jnp.zeros((_GP, _SW - 2), jnp.float32)], axis=1)

    @pl.when(i >= nb)
    def _apply():
        s0 = s0_scr[pl.ds(blk * _B, _B), :].astype(jnp.float32)
        vrow = v_scr[pl.ds(blk * _B, _B), :, :]
        gmean = jnp.dot(onehot, gath[:, 0:_C],
                        precision=lax.Precision.HIGHEST,
                        preferred_element_type=jnp.float32)
        stats = jnp.dot(onehot, gath[:, _C:],
                        preferred_element_type=jnp.float32)
        inv_std = stats[:, 0:1]
        inv_vn = stats[:, 1:2]
        sout_ref[...] = ((s0 - gmean) * inv_std * w_ref[0, :][None, :]
                         + b_ref[0, :][None, :])
        vout_ref[...] = vrow.astype(jnp.float32) * inv_vn[:, :, None]


def kernel(s, v, splits, weight, bias):
    N, C = s.shape
    G = splits.shape[0]
    nb = (N + _B - 1) // _B

    ends = jnp.cumsum(splits.astype(jnp.int32))
    starts = ends - splits.astype(jnp.int32)
    big = jnp.int32(2 ** 30)
    ends_p = jnp.pad(ends, (0, _GP - G),
                     constant_values=jnp.int32(N)).reshape(1, _GP)
    starts_p = jnp.pad(starts, (0, _GP - G),
                       constant_values=big).reshape(1, _GP)
    splits_p = jnp.pad(splits.astype(jnp.int32), (0, _GP - G)).reshape(1, _GP)
    w2 = weight.astype(jnp.float32).reshape(1, C)
    b2 = bias.astype(jnp.float32).reshape(1, C)

    full = lambda shape: pl.BlockSpec(shape, lambda i: (0,) * len(shape))
    in_idx = lambda i: jnp.where(i < nb, i, 0)
    out_idx = lambda i: jnp.where(i < nb, 0, i - nb)
    rows2 = pl.BlockSpec((_B, _C), lambda i: (in_idx(i), 0))
    rows3 = pl.BlockSpec((_B, 3, _C), lambda i: (in_idx(i), 0, 0))
    orow2 = pl.BlockSpec((_B, _C), lambda i: (out_idx(i), 0))
    orow3 = pl.BlockSpec((_B, 3, _C), lambda i: (out_idx(i), 0, 0))

    sout, vout = pl.pallas_call(
        _fused_kernel,
        grid=(2 * nb,),
        in_specs=[full((1, _GP)), full((1, _GP)), full((1, _GP)),
                  rows2, rows3, full((1, _C)), full((1, _C))],
        out_specs=[orow2, orow3],
        out_shape=[jax.ShapeDtypeStruct((N, _C), jnp.float32),
                   jax.ShapeDtypeStruct((N, 3, _C), jnp.float32)],
        scratch_shapes=[pltpu.VMEM((_GP, _C), jnp.float32),
                        pltpu.VMEM((_GP, _C), jnp.float32),
                        pltpu.VMEM((_GP, _C), jnp.float32),
                        pltpu.VMEM((_GP, _C + _SW), jnp.float32),
                        pltpu.VMEM((nb * _B, _C), jnp.bfloat16),
                        pltpu.VMEM((nb * _B, 3, _C), jnp.bfloat16)],
        compiler_params=pltpu.CompilerParams(
            dimension_semantics=("arbitrary",)),
    )(starts_p, ends_p, splits_p, s, v, w2, b2)

    return sout, vout


# speedup vs baseline: 1.1848x; 1.1848x over previous
import jax
import jax.numpy as jnp
from jax import lax
from jax.experimental import pallas as pl
from jax.experimental.pallas import tpu as pltpu

EPS = 1e-6
_B = 512
_C = 256
_GP = 256
_SW = 128


def _row_mean(srow):
    return jnp.mean(srow, axis=1, keepdims=True)


def _seg_onehot(starts, ends, i):
    r = i * _B + lax.broadcasted_iota(jnp.int32, (_B, _GP), 0)
    return ((r >= starts[None, :]) & (r < ends[None, :])).astype(jnp.float32)


def _fused_kernel(starts_ref, ends_ref, splits_ref, s_ref, v_ref,
                  w_ref, b_ref, sout_ref, vout_ref,
                  s1_acc, s2_acc, vn_acc, gath, s0_scr, v_scr):
    i = pl.program_id(0)
    nb = pl.num_programs(0) // 2
    blk = jnp.where(i < nb, i, i - nb)
    onehot = _seg_onehot(starts_ref[0, :], ends_ref[0, :], blk)

    @pl.when(i == 0)
    def _init():
        s1_acc[...] = jnp.zeros_like(s1_acc)
        s2_acc[...] = jnp.zeros_like(s2_acc)
        vn_acc[...] = jnp.zeros_like(vn_acc)

    @pl.when(i < nb)
    def _stats():
        valid = (blk * _B + lax.broadcasted_iota(jnp.int32, (_B, 1), 0)) < \
            ends_ref[0, _GP - 1]
        srow = s_ref[...]
        s0 = jnp.where(valid, srow - _row_mean(srow), 0.0)
        s0 = s0.astype(jnp.bfloat16).astype(jnp.float32)
        vrow = v_ref[...]
        vsq = vrow * vrow
        vnmat = jnp.sqrt(vsq[:, 0:_C] + vsq[:, _C:2 * _C]
                         + vsq[:, 2 * _C:3 * _C] + EPS)
        vnmat = jnp.where(valid, vnmat, 0.0)
        s0_scr[pl.ds(blk * _B, _B), :] = s0.astype(jnp.bfloat16)
        v_scr[pl.ds(blk * _B, _B), :] = vrow.astype(jnp.bfloat16)
        dn = (((0,), (0,)), ((), ()))
        s1_acc[...] += lax.dot_general(
            onehot, s0, dn, precision=lax.Precision.HIGHEST,
            preferred_element_type=jnp.float32)
        s2_acc[...] += lax.dot_general(
            onehot, s0 * s0, dn, preferred_element_type=jnp.float32)
        vn_acc[...] += lax.dot_general(
            onehot, vnmat, dn, preferred_element_type=jnp.float32)

    @pl.when(i == nb - 1)
    def _finalize():
        counts = jnp.maximum(splits_ref[0, :], 1).astype(jnp.float32)
        means = s1_acc[...] / counts[:, None]
        var = (jnp.sum(s2_acc[...], axis=1) / counts
               - jnp.sum(means * means, axis=1)) / _C
        inv_std = 1.0 / jnp.sqrt(jnp.maximum(var, 0.0) + EPS)
        vnorm = jnp.sum(vn_acc[...], axis=1) / (counts * _C)
        inv_vn = jnp.where(vnorm > 0, 1.0 / vnorm, 0.0)
        gath[:, 0:_C] = means
        gath[:, _C:] = jnp.concatenate(
            [inv_std[:, None], inv_vn[:, None],
             jnp.zeros((_GP, _SW - 2), jnp.float32)], axis=1)

    @pl.when(i >= nb)
    def _apply():
        s0 = s0_scr[pl.ds(blk * _B, _B), :].astype(jnp.float32)
        vrow = v_scr[pl.ds(blk * _B, _B), :]
        gmean = jnp.dot(onehot, gath[:, 0:_C],
                        precision=lax.Precision.HIGHEST,
                        preferred_element_type=jnp.float32)
        stats = jnp.dot(onehot, gath[:, _C:],
                        preferred_element_type=jnp.float32)
        inv_std = stats[:, 0:1]
        inv_vn = stats[:, 1:2]
        sout_ref[...] = ((s0 - gmean) * inv_std * w_ref[0, :][None, :]
                         + b_ref[0, :][None, :])
        vout_ref[...] = vrow.astype(jnp.float32) * inv_vn


def kernel(s, v, splits, weight, bias):
    N, C = s.shape
    G = splits.shape[0]
    nb = (N + _B - 1) // _B

    ends = jnp.cumsum(splits.astype(jnp.int32))
    starts = ends - splits.astype(jnp.int32)
    big = jnp.int32(2 ** 30)
    ends_p = jnp.pad(ends, (0, _GP - G),
                     constant_values=jnp.int32(N)).reshape(1, _GP)
    starts_p = jnp.pad(starts, (0, _GP - G),
                       constant_values=big).reshape(1, _GP)
    splits_p = jnp.pad(splits.astype(jnp.int32), (0, _GP - G)).reshape(1, _GP)
    w2 = weight.astype(jnp.float32).reshape(1, C)
    b2 = bias.astype(jnp.float32).reshape(1, C)

    full = lambda shape: pl.BlockSpec(shape, lambda i: (0,) * len(shape))
    in_idx = lambda i: jnp.where(i < nb, i, 0)
    out_idx = lambda i: jnp.where(i < nb, 0, i - nb)
    rows2 = pl.BlockSpec((_B, _C), lambda i: (in_idx(i), 0))
    rows3 = pl.BlockSpec((_B, 3 * _C), lambda i: (in_idx(i), 0))
    orow2 = pl.BlockSpec((_B, _C), lambda i: (out_idx(i), 0))
    orow3 = pl.BlockSpec((_B, 3 * _C), lambda i: (out_idx(i), 0))

    v2 = v.reshape(N, 3 * C)
    sout, vout = pl.pallas_call(
        _fused_kernel,
        grid=(2 * nb,),
        in_specs=[full((1, _GP)), full((1, _GP)), full((1, _GP)),
                  rows2, rows3, full((1, _C)), full((1, _C))],
        out_specs=[orow2, orow3],
        out_shape=[jax.ShapeDtypeStruct((N, _C), jnp.float32),
                   jax.ShapeDtypeStruct((N, 3 * _C), jnp.float32)],
        scratch_shapes=[pltpu.VMEM((_GP, _C), jnp.float32),
                        pltpu.VMEM((_GP, _C), jnp.float32),
                        pltpu.VMEM((_GP, _C), jnp.float32),
                        pltpu.VMEM((_GP, _C + _SW), jnp.float32),
                        pltpu.VMEM((nb * _B, _C), jnp.bfloat16),
                        pltpu.VMEM((nb * _B, 3 * _C), jnp.bfloat16)],
        compiler_params=pltpu.CompilerParams(
            dimension_semantics=("arbitrary",)),
    )(starts_p, ends_p, splits_p, s, v2, w2, b2)

    return sout, vout.reshape(N, 3, C)


# speedup vs baseline: 3.5050x; 2.9583x over previous
import jax
import jax.numpy as jnp
from jax import lax
from jax.experimental import pallas as pl
from jax.experimental.pallas import tpu as pltpu

EPS = 1e-6
_B = 512
_C = 256
_GP = 256
_SW = 128


def _row_mean(srow):
    return jnp.mean(srow, axis=1, keepdims=True)


def _seg_onehot(starts, ends, i):
    r = i * _B + lax.broadcasted_iota(jnp.int32, (_B, _GP), 0)
    return ((r >= starts[None, :]) & (r < ends[None, :])).astype(jnp.float32)


def _fused_kernel(starts_ref, ends_ref, splits_ref, s_ref, v_ref,
                  w_ref, b_ref, sout_ref, vout_ref,
                  s1_acc, s2_acc, vn_acc, gath, s0_scr, v_scr):
    i = pl.program_id(0)
    nb = pl.num_programs(0) // 2
    blk = jnp.where(i < nb, i, i - nb)
    onehot = _seg_onehot(starts_ref[0, :], ends_ref[0, :], blk)

    @pl.when(i == 0)
    def _init():
        s1_acc[...] = jnp.zeros_like(s1_acc)
        s2_acc[...] = jnp.zeros_like(s2_acc)
        vn_acc[...] = jnp.zeros_like(vn_acc)

    @pl.when(i < nb)
    def _stats():
        valid = (blk * _B + lax.broadcasted_iota(jnp.int32, (_B, 1), 0)) < \
            ends_ref[0, _GP - 1]
        srow = s_ref[...]
        s0 = jnp.where(valid, srow - _row_mean(srow), 0.0)
        s0 = s0.astype(jnp.bfloat16).astype(jnp.float32)
        vrow = v_ref[...]
        vnmat = jnp.sqrt(vrow[0] * vrow[0] + vrow[1] * vrow[1]
                         + vrow[2] * vrow[2] + EPS)
        vnmat = jnp.where(valid, vnmat, 0.0)
        s0_scr[pl.ds(blk * _B, _B), :] = s0.astype(jnp.bfloat16)
        v_scr[:, pl.ds(blk * _B, _B), :] = vrow.astype(jnp.bfloat16)
        dn = (((0,), (0,)), ((), ()))
        s1_acc[...] += lax.dot_general(
            onehot, s0, dn, precision=lax.Precision.HIGHEST,
            preferred_element_type=jnp.float32)
        s2_acc[...] += lax.dot_general(
            onehot, s0 * s0, dn, preferred_element_type=jnp.float32)
        vn_acc[...] += lax.dot_general(
            onehot, vnmat, dn, preferred_element_type=jnp.float32)

    @pl.when(i == nb - 1)
    def _finalize():
        counts = jnp.maximum(splits_ref[0, :], 1).astype(jnp.float32)
        means = s1_acc[...] / counts[:, None]
        var = (jnp.sum(s2_acc[...], axis=1) / counts
               - jnp.sum(means * means, axis=1)) / _C
        inv_std = 1.0 / jnp.sqrt(jnp.maximum(var, 0.0) + EPS)
        vnorm = jnp.sum(vn_acc[...], axis=1) / (counts * _C)
        inv_vn = jnp.where(vnorm > 0, 1.0 / vnorm, 0.0)
        gath[:, 0:_C] = means
        gath[:, _C:] = jnp.concatenate(
            [inv_std[:, None], inv_vn[:, None],
             jnp.zeros((_GP, _SW - 2), jnp.float32)], axis=1)

    @pl.when(i >= nb)
    def _apply():
        s0 = s0_scr[pl.ds(blk * _B, _B), :].astype(jnp.float32)
        vrow = v_scr[:, pl.ds(blk * _B, _B), :]
        gmean = jnp.dot(onehot, gath[:, 0:_C],
                        precision=lax.Precision.HIGHEST,
                        preferred_element_type=jnp.float32)
        stats = jnp.dot(onehot, gath[:, _C:],
                        preferred_element_type=jnp.float32)
        inv_std = stats[:, 0:1]
        inv_vn = stats[:, 1:2]
        sout_ref[...] = ((s0 - gmean) * inv_std * w_ref[0, :][None, :]
                         + b_ref[0, :][None, :])
        vout_ref[...] = vrow.astype(jnp.float32) * inv_vn[None]


def kernel(s, v, splits, weight, bias):
    N, C = s.shape
    G = splits.shape[0]
    nb = (N + _B - 1) // _B

    ends = jnp.cumsum(splits.astype(jnp.int32))
    starts = ends - splits.astype(jnp.int32)
    big = jnp.int32(2 ** 30)
    ends_p = jnp.pad(ends, (0, _GP - G),
                     constant_values=jnp.int32(N)).reshape(1, _GP)
    starts_p = jnp.pad(starts, (0, _GP - G),
                       constant_values=big).reshape(1, _GP)
    splits_p = jnp.pad(splits.astype(jnp.int32), (0, _GP - G)).reshape(1, _GP)
    w2 = weight.astype(jnp.float32).reshape(1, C)
    b2 = bias.astype(jnp.float32).reshape(1, C)

    full = lambda shape: pl.BlockSpec(shape, lambda i: (0,) * len(shape))
    in_idx = lambda i: jnp.where(i < nb, i, 0)
    out_idx = lambda i: jnp.where(i < nb, 0, i - nb)
    rows2 = pl.BlockSpec((_B, _C), lambda i: (in_idx(i), 0))
    rows3 = pl.BlockSpec((3, _B, _C), lambda i: (0, in_idx(i), 0))
    orow2 = pl.BlockSpec((_B, _C), lambda i: (out_idx(i), 0))
    orow3 = pl.BlockSpec((3, _B, _C), lambda i: (0, out_idx(i), 0))

    vt = jnp.transpose(v, (1, 0, 2))
    sout, vout = pl.pallas_call(
        _fused_kernel,
        grid=(2 * nb,),
        in_specs=[full((1, _GP)), full((1, _GP)), full((1, _GP)),
                  rows2, rows3, full((1, _C)), full((1, _C))],
        out_specs=[orow2, orow3],
        out_shape=[jax.ShapeDtypeStruct((N, _C), jnp.float32),
                   jax.ShapeDtypeStruct((3, N, _C), jnp.float32)],
        scratch_shapes=[pltpu.VMEM((_GP, _C), jnp.float32),
                        pltpu.VMEM((_GP, _C), jnp.float32),
                        pltpu.VMEM((_GP, _C), jnp.float32),
                        pltpu.VMEM((_GP, _C + _SW), jnp.float32),
                        pltpu.VMEM((nb * _B, _C), jnp.bfloat16),
                        pltpu.VMEM((3, nb * _B, _C), jnp.bfloat16)],
        compiler_params=pltpu.CompilerParams(
            dimension_semantics=("arbitrary",)),
    )(starts_p, ends_p, splits_p, s, vt, w2, b2)

    return sout, jnp.transpose(vout, (1, 0, 2))


# speedup vs baseline: 4.5437x; 1.2963x over previous
import jax
import jax.numpy as jnp
from jax import lax
from jax.experimental import pallas as pl
from jax.experimental.pallas import tpu as pltpu

EPS = 1e-6
_B = 1024
_C = 256
_GP = 256
_SW = 128


def _row_mean(srow):
    return jnp.mean(srow, axis=1, keepdims=True)


def _seg_onehot(starts, ends, i):
    r = i * _B + lax.broadcasted_iota(jnp.int32, (_B, _GP), 0)
    return ((r >= starts[None, :]) & (r < ends[None, :])).astype(jnp.float32)


def _fused_kernel(starts_ref, ends_ref, splits_ref, s_ref, v_ref,
                  w_ref, b_ref, sout_ref, vout_ref,
                  s1_acc, s2_acc, vn_acc, gath, s0_scr, v_scr):
    i = pl.program_id(0)
    nb = pl.num_programs(0) // 2
    blk = jnp.where(i < nb, i, i - nb)
    onehot = _seg_onehot(starts_ref[0, :], ends_ref[0, :], blk)

    @pl.when(i == 0)
    def _init():
        s1_acc[...] = jnp.zeros_like(s1_acc)
        s2_acc[...] = jnp.zeros_like(s2_acc)
        vn_acc[...] = jnp.zeros_like(vn_acc)

    @pl.when(i < nb)
    def _stats():
        valid = (blk * _B + lax.broadcasted_iota(jnp.int32, (_B, 1), 0)) < \
            ends_ref[0, _GP - 1]
        srow = s_ref[...]
        s0 = jnp.where(valid, srow - _row_mean(srow), 0.0)
        s0 = s0.astype(jnp.bfloat16).astype(jnp.float32)
        vrow = v_ref[...]
        vnmat = jnp.sqrt(vrow[0] * vrow[0] + vrow[1] * vrow[1]
                         + vrow[2] * vrow[2] + EPS)
        vnmat = jnp.where(valid, vnmat, 0.0)
        s0_scr[pl.ds(blk * _B, _B), :] = s0.astype(jnp.bfloat16)
        v_scr[:, pl.ds(blk * _B, _B), :] = vrow.astype(jnp.bfloat16)
        dn = (((0,), (0,)), ((), ()))
        s1_acc[...] += lax.dot_general(
            onehot, s0, dn, preferred_element_type=jnp.float32)
        s2_acc[...] += lax.dot_general(
            onehot, s0 * s0, dn, preferred_element_type=jnp.float32)
        vn_acc[...] += lax.dot_general(
            onehot, vnmat, dn, preferred_element_type=jnp.float32)

    @pl.when(i == nb - 1)
    def _finalize():
        counts = jnp.maximum(splits_ref[0, :], 1).astype(jnp.float32)
        means = s1_acc[...] / counts[:, None]
        var = (jnp.sum(s2_acc[...], axis=1) / counts
               - jnp.sum(means * means, axis=1)) / _C
        inv_std = 1.0 / jnp.sqrt(jnp.maximum(var, 0.0) + EPS)
        vnorm = jnp.sum(vn_acc[...], axis=1) / (counts * _C)
        inv_vn = jnp.where(vnorm > 0, 1.0 / vnorm, 0.0)
        gath[:, 0:_C] = means
        gath[:, _C:] = jnp.concatenate(
            [inv_std[:, None], inv_vn[:, None],
             jnp.zeros((_GP, _SW - 2), jnp.float32)], axis=1)

    @pl.when(i >= nb)
    def _apply():
        s0 = s0_scr[pl.ds(blk * _B, _B), :].astype(jnp.float32)
        vrow = v_scr[:, pl.ds(blk * _B, _B), :]
        gmean = jnp.dot(onehot, gath[:, 0:_C],
                        precision=lax.Precision.HIGHEST,
                        preferred_element_type=jnp.float32)
        stats = jnp.dot(onehot, gath[:, _C:],
                        preferred_element_type=jnp.float32)
        inv_std = stats[:, 0:1]
        inv_vn = stats[:, 1:2]
        sout_ref[...] = ((s0 - gmean) * inv_std * w_ref[0, :][None, :]
                         + b_ref[0, :][None, :])
        vout_ref[...] = vrow.astype(jnp.float32) * inv_vn[None]


def kernel(s, v, splits, weight, bias):
    N, C = s.shape
    G = splits.shape[0]
    nb = (N + _B - 1) // _B

    ends = jnp.cumsum(splits.astype(jnp.int32))
    starts = ends - splits.astype(jnp.int32)
    big = jnp.int32(2 ** 30)
    ends_p = jnp.pad(ends, (0, _GP - G),
                     constant_values=jnp.int32(N)).reshape(1, _GP)
    starts_p = jnp.pad(starts, (0, _GP - G),
                       constant_values=big).reshape(1, _GP)
    splits_p = jnp.pad(splits.astype(jnp.int32), (0, _GP - G)).reshape(1, _GP)
    w2 = weight.astype(jnp.float32).reshape(1, C)
    b2 = bias.astype(jnp.float32).reshape(1, C)

    full = lambda shape: pl.BlockSpec(shape, lambda i: (0,) * len(shape))
    in_idx = lambda i: jnp.where(i < nb, i, 0)
    out_idx = lambda i: jnp.where(i < nb, 0, i - nb)
    rows2 = pl.BlockSpec((_B, _C), lambda i: (in_idx(i), 0))
    rows3 = pl.BlockSpec((3, _B, _C), lambda i: (0, in_idx(i), 0))
    orow2 = pl.BlockSpec((_B, _C), lambda i: (out_idx(i), 0))
    orow3 = pl.BlockSpec((3, _B, _C), lambda i: (0, out_idx(i), 0))

    vt = jnp.transpose(v, (1, 0, 2))
    sout, vout = pl.pallas_call(
        _fused_kernel,
        grid=(2 * nb,),
        in_specs=[full((1, _GP)), full((1, _GP)), full((1, _GP)),
                  rows2, rows3, full((1, _C)), full((1, _C))],
        out_specs=[orow2, orow3],
        out_shape=[jax.ShapeDtypeStruct((N, _C), jnp.float32),
                   jax.ShapeDtypeStruct((3, N, _C), jnp.float32)],
        scratch_shapes=[pltpu.VMEM((_GP, _C), jnp.float32),
                        pltpu.VMEM((_GP, _C), jnp.float32),
                        pltpu.VMEM((_GP, _C), jnp.float32),
                        pltpu.VMEM((_GP, _C + _SW), jnp.float32),
                        pltpu.VMEM((nb * _B, _C), jnp.bfloat16),
                        pltpu.VMEM((3, nb * _B, _C), jnp.bfloat16)],
        compiler_params=pltpu.CompilerParams(
            dimension_semantics=("arbitrary",)),
    )(starts_p, ends_p, splits_p, s, vt, w2, b2)

    return sout, jnp.transpose(vout, (1, 0, 2))


# speedup vs baseline: 4.8966x; 1.0777x over previous
import jax
import jax.numpy as jnp
from jax import lax
from jax.experimental import pallas as pl
from jax.experimental.pallas import tpu as pltpu

EPS = 1e-6
_B = 1024
_C = 256
_GP = 256
_SW = 128


def _row_mean(srow):
    return jnp.mean(srow, axis=1, keepdims=True)


def _seg_onehot(starts, ends, i):
    r = i * _B + lax.broadcasted_iota(jnp.int32, (_B, _GP), 0)
    return ((r >= starts[None, :]) & (r < ends[None, :])).astype(jnp.float32)


def _fused_kernel(starts_ref, ends_ref, splits_ref, s_ref, v_ref,
                  w_ref, b_ref, sout_ref, vout_ref,
                  s1_acc, s2_acc, vn_acc, gath, s0_scr, v_scr):
    i = pl.program_id(0)
    nb = pl.num_programs(0) // 2
    blk = jnp.where(i < nb, i, i - nb)
    onehot = _seg_onehot(starts_ref[0, :], ends_ref[0, :], blk)

    @pl.when(i == 0)
    def _init():
        s1_acc[...] = jnp.zeros_like(s1_acc)
        s2_acc[...] = jnp.zeros_like(s2_acc)
        vn_acc[...] = jnp.zeros_like(vn_acc)

    @pl.when(i < nb)
    def _stats():
        valid = (blk * _B + lax.broadcasted_iota(jnp.int32, (_B, 1), 0)) < \
            ends_ref[0, _GP - 1]
        srow = s_ref[...]
        s0 = jnp.where(valid, srow - _row_mean(srow), 0.0)
        s0 = s0.astype(jnp.bfloat16).astype(jnp.float32)
        vrow = v_ref[...]
        vnmat = jnp.sqrt(vrow[0] * vrow[0] + vrow[1] * vrow[1]
                         + vrow[2] * vrow[2] + EPS)
        vnmat = jnp.where(valid, vnmat, 0.0)
        s0_scr[pl.ds(blk * _B, _B), :] = s0.astype(jnp.bfloat16)
        v_scr[:, pl.ds(blk * _B, _B), :] = vrow.astype(jnp.bfloat16)
        dn = (((0,), (0,)), ((), ()))
        s1_acc[...] += lax.dot_general(
            onehot, s0, dn, preferred_element_type=jnp.float32)
        s2_acc[...] += lax.dot_general(
            onehot, s0 * s0, dn, preferred_element_type=jnp.float32)
        vn_acc[...] += lax.dot_general(
            onehot, vnmat, dn, preferred_element_type=jnp.float32)

    @pl.when(i == nb - 1)
    def _finalize():
        counts = jnp.maximum(splits_ref[0, :], 1).astype(jnp.float32)
        means = s1_acc[...] / counts[:, None]
        var = (jnp.sum(s2_acc[...], axis=1) / counts
               - jnp.sum(means * means, axis=1)) / _C
        inv_std = 1.0 / jnp.sqrt(jnp.maximum(var, 0.0) + EPS)
        vnorm = jnp.sum(vn_acc[...], axis=1) / (counts * _C)
        inv_vn = jnp.where(vnorm > 0, 1.0 / vnorm, 0.0)
        m_hi = means.astype(jnp.bfloat16).astype(jnp.float32)
        gath[:, 0:_C] = m_hi
        gath[:, _C:2 * _C] = means - m_hi
        gath[:, 2 * _C:] = jnp.concatenate(
            [inv_std[:, None], inv_vn[:, None],
             jnp.zeros((_GP, _SW - 2), jnp.float32)], axis=1)

    @pl.when(i >= nb)
    def _apply():
        s0 = s0_scr[pl.ds(blk * _B, _B), :].astype(jnp.float32)
        vrow = v_scr[:, pl.ds(blk * _B, _B), :]
        gmean = (jnp.dot(onehot, gath[:, 0:_C],
                         preferred_element_type=jnp.float32)
                 + jnp.dot(onehot, gath[:, _C:2 * _C],
                           preferred_element_type=jnp.float32))
        stats = jnp.dot(onehot, gath[:, 2 * _C:],
                        preferred_element_type=jnp.float32)
        inv_std = stats[:, 0:1]
        inv_vn = stats[:, 1:2]
        sout_ref[...] = ((s0 - gmean) * inv_std * w_ref[0, :][None, :]
                         + b_ref[0, :][None, :])
        vout_ref[...] = vrow.astype(jnp.float32) * inv_vn[None]


def kernel(s, v, splits, weight, bias):
    N, C = s.shape
    G = splits.shape[0]
    nb = (N + _B - 1) // _B

    ends = jnp.cumsum(splits.astype(jnp.int32))
    starts = ends - splits.astype(jnp.int32)
    big = jnp.int32(2 ** 30)
    ends_p = jnp.pad(ends, (0, _GP - G),
                     constant_values=jnp.int32(N)).reshape(1, _GP)
    starts_p = jnp.pad(starts, (0, _GP - G),
                       constant_values=big).reshape(1, _GP)
    splits_p = jnp.pad(splits.astype(jnp.int32), (0, _GP - G)).reshape(1, _GP)
    w2 = weight.astype(jnp.float32).reshape(1, C)
    b2 = bias.astype(jnp.float32).reshape(1, C)

    full = lambda shape: pl.BlockSpec(shape, lambda i: (0,) * len(shape))
    in_idx = lambda i: jnp.where(i < nb, i, 0)
    out_idx = lambda i: jnp.where(i < nb, 0, i - nb)
    rows2 = pl.BlockSpec((_B, _C), lambda i: (in_idx(i), 0))
    rows3 = pl.BlockSpec((3, _B, _C), lambda i: (0, in_idx(i), 0))
    orow2 = pl.BlockSpec((_B, _C), lambda i: (out_idx(i), 0))
    orow3 = pl.BlockSpec((3, _B, _C), lambda i: (0, out_idx(i), 0))

    vt = jnp.transpose(v, (1, 0, 2))
    sout, vout = pl.pallas_call(
        _fused_kernel,
        grid=(2 * nb,),
        in_specs=[full((1, _GP)), full((1, _GP)), full((1, _GP)),
                  rows2, rows3, full((1, _C)), full((1, _C))],
        out_specs=[orow2, orow3],
        out_shape=[jax.ShapeDtypeStruct((N, _C), jnp.float32),
                   jax.ShapeDtypeStruct((3, N, _C), jnp.float32)],
        scratch_shapes=[pltpu.VMEM((_GP, _C), jnp.float32),
                        pltpu.VMEM((_GP, _C), jnp.float32),
                        pltpu.VMEM((_GP, _C), jnp.float32),
                        pltpu.VMEM((_GP, 2 * _C + _SW), jnp.float32),
                        pltpu.VMEM((nb * _B, _C), jnp.bfloat16),
                        pltpu.VMEM((3, nb * _B, _C), jnp.bfloat16)],
        compiler_params=pltpu.CompilerParams(
            dimension_semantics=("arbitrary",)),
    )(starts_p, ends_p, splits_p, s, vt, w2, b2)

    return sout, jnp.transpose(vout, (1, 0, 2))
